# baseline (speedup 1.0000x reference)
import jax
import jax.numpy as jnp
from jax.experimental import pallas as pl

S = 10
NB = 16
NODE1 = 128
DEG = 8
BS = 2


def _nrm(x, axis):
    n = jnp.linalg.norm(x, axis=axis, keepdims=True)
    return x / jnp.maximum(n, 1e-12)


def _knn_idx(vertices, neighbor_num):
    inner = jnp.einsum('bik,bjk->bij', vertices, vertices)
    quadratic = jnp.sum(vertices ** 2, axis=2)
    distance = inner * -2 + quadratic[:, None, :] + quadratic[:, :, None]
    _, idx = jax.lax.top_k(-distance, neighbor_num + 1)
    return idx[:, :, 1:]


def _gather_nb(tensor, index):
    return jax.vmap(lambda t, i: t[i])(tensor, index)


def _nbr_dir_norm(vertices, index):
    neighbors = _gather_nb(vertices, index)
    direction = neighbors - vertices[:, :, None, :]
    return _nrm(direction, -1)


def _conv_surface(index, vertices, directions, kernel_num):
    bs, v, n = index.shape
    nd = _nbr_dir_norm(vertices, index)
    sd = _nrm(directions, 0)
    theta = jax.nn.relu(nd @ sd)
    theta = theta.reshape(bs, v, n, S, kernel_num)
    return jnp.sum(jnp.max(theta, axis=2), axis=2)


def _conv_layer(index, vertices, fmap, weights, bias, directions, out_channel):
    bs, v, n = index.shape
    nd = _nbr_dir_norm(vertices, index)
    sd = _nrm(directions, 0)
    theta = jax.nn.relu(nd @ sd)
    fout = fmap @ weights + bias
    fc = fout[:, :, :out_channel]
    fs = _gather_nb(fout[:, :, out_channel:], index)
    act = (theta * fs).reshape(bs, v, n, S, out_channel)
    act = jnp.sum(jnp.max(act, axis=2), axis=2)
    return fc + act


def _tree_gcn(x, w_root, w_branch, w_loop1, w_loop2, bias, node, out_feature, activation):
    bs, root_num, in_feature = x.shape
    repeat_num = node // root_num
    root_node = x @ w_root
    root = jnp.tile(root_node, (1, 1, repeat_num)).reshape(bs, -1, out_feature)
    branch = jnp.einsum('bni,nio->bno', x, w_branch)
    branch = jax.nn.leaky_relu(branch, 0.2)
    branch = branch.reshape(bs, node * DEG, in_feature)
    branch = (branch @ w_loop1) @ w_loop2
    root_rep = jnp.tile(root, (1, 1, DEG)).reshape(bs, -1, out_feature)
    branch = root_rep + branch
    if activation:
        branch = jax.nn.leaky_relu(branch + jnp.tile(bias, (1, node, 1)), 0.2)
    return branch


def _add_kernel(a_ref, b_ref, o_ref):
    o_ref[...] = a_ref[...] + b_ref[...]


def _pallas_add(a, b):
    total = a.shape[0] * a.shape[1] * a.shape[2]
    a2 = a.reshape(8, total // 8)
    b2 = b.reshape(8, total // 8)
    out = pl.pallas_call(
        _add_kernel,
        out_shape=jax.ShapeDtypeStruct((8, total // 8), a.dtype),
    )(a2, b2)
    return out.reshape(a.shape)


def _tree_gcn2(x, w_root, w_branch, w_loop1, w_loop2, node):
    bs, root_num, in_feature = x.shape
    root = x @ w_root
    branch = jnp.einsum('bni,nio->bno', x, w_branch)
    branch = jax.nn.leaky_relu(branch, 0.2)
    branch = branch.reshape(bs, node * DEG, in_feature)
    branch = (branch @ w_loop1) @ w_loop2
    root_rep = jnp.tile(root, (1, 1, DEG)).reshape(bs, -1, 3)
    return _pallas_add(root_rep, branch)


def kernel(coase_points, dir0, w1, b1, dir1, dir2, w3, b3, dir3, w4, b4, dir4, w5, b5, dir5, wr1, wb1, wl1a, wl1b, tb1, wr2, wb2, wl2a, wl2b):
    ni = _knn_idx(coase_points, NB)
    fm0 = jax.nn.relu(_conv_surface(ni, coase_points, dir0, 32))
    fm1 = jax.nn.relu(_conv_layer(ni, coase_points, fm0, w1, b1, dir1, 64))
    pc1 = _tree_gcn(fm1, wr1, wb1, wl1a, wl1b, tb1, NODE1, 3, True)
    ni2 = _knn_idx(pc1, NB)
    fm2 = jax.nn.relu(_conv_surface(ni2, pc1, dir2, 32))
    fm3 = jax.nn.relu(_conv_layer(ni2, pc1, fm2, w3, b3, dir3, 64))
    fm4 = jax.nn.relu(_conv_layer(ni2, pc1, fm3, w4, b4, dir4, 64))
    fm5 = jax.nn.relu(_conv_layer(ni2, pc1, fm4, w5, b5, dir5, 64))
    pc2 = _tree_gcn2(fm5, wr2, wb2, wl2a, wl2b, NODE1 * DEG)
    return pc1, pc2


# speedup vs baseline: 4.8870x; 4.8870x over previous
import functools

import jax
import jax.numpy as jnp
from jax.experimental import pallas as pl
from jax.experimental.pallas import tpu as pltpu
from jax.experimental.pallas import tpu_sc as plsc

S = 10
NB = 16
NODE1 = 128
DEG = 8
BS = 2

_F32 = jnp.float32
_HIGH = jax.lax.Precision.HIGHEST


def _dot(a, b, precision=None):
    return jnp.dot(a, b, preferred_element_type=_F32, precision=precision)


def _dot_bf16(a, b):
    return jnp.dot(a.astype(jnp.bfloat16), b.astype(jnp.bfloat16),
                   preferred_element_type=_F32)


def _relu(x):
    return jnp.maximum(x, 0.0)


def _lrelu(x):
    return jnp.where(x >= 0, x, 0.2 * x)


def _norm_cols(d):
    n = jnp.sqrt(jnp.sum(d * d, axis=0, keepdims=True))
    return d / jnp.maximum(n, 1e-12)


def _extract_knn(dmat, iota_lane):
    picks = []
    d_cur = dmat
    for t in range(NB + 1):
        m = jnp.min(d_cur, axis=1, keepdims=True)
        eq = d_cur == m
        idxf = jnp.min(jnp.where(eq, iota_lane, 1073741824.0), axis=1,
                       keepdims=True)
        oh_b = iota_lane == idxf
        if t > 0:
            picks.append((oh_b.astype(_F32), idxf))
        d_cur = jnp.where(oh_b, jnp.inf, d_cur)
    return picks


def _nd_from_onehot(oh, pts):
    nbr = _dot(oh, pts, precision=_HIGH)
    dirv = nbr - pts
    n = jnp.sqrt(jnp.sum(dirv * dirv, axis=1, keepdims=True))
    return dirv / jnp.maximum(n, 1e-12)


def _sum_s(x, w):
    acc = x[:, 0:w]
    for s in range(1, S):
        acc = acc + x[:, s * w:(s + 1) * w]
    return acc


def _stage1_body(pts_ref, ptsT_ref, dir0_ref, w1_ref, b1_ref, dir1_ref,
                 wr1_ref, wb1_ref, wl1a_ref, wl1b_ref, tb1_ref, out_ref):
    pts = pts_ref[0]
    ptsT = ptsT_ref[0]
    g = _dot_bf16(pts, ptsT)
    q = jnp.sum(pts * pts, axis=1, keepdims=True)
    qT = jnp.sum(ptsT * ptsT, axis=0, keepdims=True)
    d = (-2.0 * g + qT) + q
    iota = jax.lax.broadcasted_iota(jnp.int32, (NODE1, NODE1), 1).astype(_F32)
    picks = _extract_knn(d, iota)
    nds = [_nd_from_onehot(oh, pts) for oh, _ in picks]

    sd0 = _norm_cols(dir0_ref[...])
    mx = None
    for nd in nds:
        th = _relu(_dot_bf16(nd, sd0))
        mx = th if mx is None else jnp.maximum(mx, th)
    fm0 = _relu(_sum_s(mx, 32))

    fout = _dot_bf16(fm0, w1_ref[...]) + b1_ref[...]
    sd1 = _norm_cols(dir1_ref[...])
    fc = fout[:, :64]
    ft = fout[:, 64:]
    acc = None
    for (oh, _), nd in zip(picks, nds):
        th = _relu(_dot_bf16(nd, sd1))
        fs = _dot(oh, ft, precision=_HIGH)
        a = th * fs
        acc = a if acc is None else jnp.maximum(acc, a)
    fm1 = _relu(fc + _sum_s(acc, 64))

    root = _dot_bf16(fm1, wr1_ref[...])
    fm1r = fm1.astype(jnp.bfloat16).astype(_F32)
    br = None
    for i in range(64):
        wbi = wb1_ref[:, i, :].astype(jnp.bfloat16).astype(_F32)
        contrib = fm1r[:, i:i + 1] * wbi
        br = contrib if br is None else br + contrib
    br = _lrelu(br)
    for dd in range(DEG):
        t1 = _dot_bf16(br[:, dd * 64:(dd + 1) * 64], wl1a_ref[...])
        o3 = _dot_bf16(t1, wl1b_ref[...])
        out_ref[0, :, dd, :] = _lrelu(o3 + root + tb1_ref[dd:dd + 1, :])


def _stage1(pts, dir0, w1, b1, dir1, wr1, wb1, wl1a, wl1b, tb1):
    ptsT = jnp.transpose(pts, (0, 2, 1))
    full = lambda *shape: pl.BlockSpec(shape, lambda i: (0,) * len(shape))
    out = pl.pallas_call(
        _stage1_body,
        out_shape=jax.ShapeDtypeStruct((BS, NODE1, DEG, 3), _F32),
        grid=(BS,),
        in_specs=[
            pl.BlockSpec((1, NODE1, 3), lambda i: (i, 0, 0)),
            pl.BlockSpec((1, 3, NODE1), lambda i: (i, 0, 0)),
            full(3, S * 32),
            full(32, (S + 1) * 64),
            full(1, (S + 1) * 64),
            full(3, S * 64),
            full(64, 3),
            full(NODE1, 64, DEG * 64),
            full(64, 64 * S),
            full(64 * S, 3),
            full(DEG, 3),
        ],
        out_specs=pl.BlockSpec((1, NODE1, DEG, 3), lambda i: (i, 0, 0, 0)),
    )(pts, ptsT, dir0, w1, b1.reshape(1, -1), dir1, wr1, wb1, wl1a, wl1b,
      tb1.reshape(DEG, 3))
    return out.reshape(BS, NODE1 * DEG, 3)


def _knn2_body(pc_ref, pcT_ref, dir2_ref, ni_ref, nd_ref, fm2_ref,
               d_ref, mx_ref, acc_ref):
    v = pc_ref.shape[1]
    b = pl.program_id(0)
    pc = pc_ref[0]
    pcT = pcT_ref[0]
    g = _dot_bf16(pc, pcT)
    q = jnp.sum(pc * pc, axis=1, keepdims=True)
    qT = jnp.sum(pcT * pcT, axis=0, keepdims=True)
    iota = jax.lax.broadcasted_iota(jnp.int32, (v, v), 1).astype(_F32)
    iota16 = jax.lax.broadcasted_iota(jnp.int32, (v, NB), 1).astype(_F32)
    sd2 = _norm_cols(dir2_ref[...])

    d0 = (-2.0 * g + qT) + q
    m = jnp.min(d0, axis=1, keepdims=True)
    eq = d0 == m
    idxf = jnp.min(jnp.where(eq, iota, 1073741824.0), axis=1, keepdims=True)
    d_ref[...] = jnp.where(iota == idxf, jnp.inf, d0)
    mx_ref[...] = jnp.zeros((v, S * 32), _F32)
    acc_ref[...] = jnp.zeros((v, NB), _F32)

    def body(t, _):
        d = d_ref[...]
        m = jnp.min(d, axis=1, keepdims=True)
        eq = d == m
        idxf = jnp.min(jnp.where(eq, iota, 1073741824.0), axis=1,
                       keepdims=True)
        oh_b = iota == idxf
        oh = oh_b.astype(_F32)
        nd = _nd_from_onehot(oh, pc)
        nd_ref[0, t] = nd
        th = _relu(_dot(nd, sd2))
        mx_ref[...] = jnp.maximum(mx_ref[...], th)
        acc_ref[...] = acc_ref[...] + jnp.where(
            iota16 == t.astype(_F32), idxf, 0.0)
        d_ref[...] = jnp.where(oh_b, jnp.inf, d)
        return 0

    jax.lax.fori_loop(0, NB, body, 0)
    ni_ref[0] = acc_ref[...].astype(jnp.int32) + v * b
    fm2 = _relu(_sum_s(mx_ref[...], 32))
    fm2_ref[0] = jnp.concatenate([fm2, jnp.zeros((v, 96), _F32)], axis=1)


def _knn2(pc1, dir2):
    v = NODE1 * DEG
    pc1T = jnp.transpose(pc1, (0, 2, 1))
    ni, nd, fm2 = pl.pallas_call(
        _knn2_body,
        out_shape=(
            jax.ShapeDtypeStruct((BS, v, NB), jnp.int32),
            jax.ShapeDtypeStruct((BS, NB, v, 3), _F32),
            jax.ShapeDtypeStruct((BS, v, 128), _F32),
        ),
        grid=(BS,),
        in_specs=[
            pl.BlockSpec((1, v, 3), lambda i: (i, 0, 0)),
            pl.BlockSpec((1, 3, v), lambda i: (i, 0, 0)),
            pl.BlockSpec((3, S * 32), lambda i: (0, 0)),
        ],
        out_specs=(
            pl.BlockSpec((1, v, NB), lambda i: (i, 0, 0)),
            pl.BlockSpec((1, NB, v, 3), lambda i: (i, 0, 0, 0)),
            pl.BlockSpec((1, v, 128), lambda i: (i, 0, 0)),
        ),
        scratch_shapes=[
            pltpu.VMEM((v, v), _F32),
            pltpu.VMEM((v, S * 32), _F32),
            pltpu.VMEM((v, NB), _F32),
        ],
    )(pc1, pc1T, dir2)
    return ni, nd, fm2


def _sc_gather(table, idx_flat):
    n, c = idx_flat.shape[0], table.shape[1]
    win = 128
    mesh = plsc.VectorSubcoreMesh(core_axis_name="c", subcore_axis_name="s")
    idx2 = idx_flat.reshape(1, n)

    @functools.partial(
        pl.kernel,
        out_type=jax.ShapeDtypeStruct((n, c), table.dtype),
        mesh=mesh)
    def _gather_kernel(x_hbm, i_hbm, o_hbm):
        def body(i_vmem, o_vmem):
            pltpu.sync_copy(x_hbm.at[i_vmem.at[0]], o_vmem)

        pltpu.emit_pipeline(
            body,
            grid=(n // win,),
            in_specs=[pl.BlockSpec((1, win), index_map=lambda i: (0, i))],
            out_specs=[pl.BlockSpec((win, c), index_map=lambda i: (i, 0))],
            core_axis_name=("c", "s"),
            dimension_semantics=(pltpu.PARALLEL,),
        )(i_hbm, o_hbm)

    return _gather_kernel(table, idx2)


def _conv2_body(cin, fm_ref, g_ref, nd_ref, w_ref, b_ref, dir_ref, out_ref,
                acc_ref):
    v = fm_ref.shape[1]
    w = w_ref[...]
    bias = b_ref[...]
    sd = _norm_cols(dir_ref[...])
    fout_head = _dot(fm_ref[0][:, :cin], w[:, :64]) + bias[:, :64]
    w_tail = w[:, 64:]
    b_tail = bias[:, 64:]
    acc_ref[...] = jnp.full((v, S * 64), -jnp.inf, _F32)

    def body(t, _):
        th = _relu(_dot(nd_ref[0, t], sd))
        fs = _dot(g_ref[0, t][:, :cin], w_tail) + b_tail
        acc_ref[...] = jnp.maximum(acc_ref[...], th * fs)
        return 0

    jax.lax.fori_loop(0, NB, body, 0)
    res = _relu(fout_head + _sum_s(acc_ref[...], 64))
    out_ref[0] = jnp.concatenate([res, jnp.zeros((v, 64), _F32)], axis=1)


def _conv2(fm, gathered, nd, w, b, dirs):
    v = NODE1 * DEG
    cin = w.shape[0]
    out = pl.pallas_call(
        functools.partial(_conv2_body, cin),
        out_shape=jax.ShapeDtypeStruct((BS, v, 128), _F32),
        grid=(BS,),
        in_specs=[
            pl.BlockSpec((1, v, 128), lambda i: (i, 0, 0)),
            pl.BlockSpec((1, NB, v, 128), lambda i: (i, 0, 0, 0)),
            pl.BlockSpec((1, NB, v, 3), lambda i: (i, 0, 0, 0)),
            pl.BlockSpec((cin, (S + 1) * 64), lambda i: (0, 0)),
            pl.BlockSpec((1, (S + 1) * 64), lambda i: (0, 0)),
            pl.BlockSpec((3, S * 64), lambda i: (0, 0)),
        ],
        out_specs=pl.BlockSpec((1, v, 128), lambda i: (i, 0, 0)),
        scratch_shapes=[pltpu.VMEM((v, S * 64), _F32)],
    )(fm, gathered, nd, w, b.reshape(1, -1), dirs)
    return out


def _conv_layer2(fm, ni_flat, nd, w, b, dirs):
    v = NODE1 * DEG
    g = _sc_gather(fm.reshape(BS * v, 128), ni_flat)
    g = g.reshape(BS, NB, v, 128)
    return _conv2(fm, g, nd, w, b, dirs)


def _tgcn2_body(fm5_ref, wb2_ref, wr2_ref, wl2a_ref, wl2b_ref, out_ref):
    wlf = _dot(wl2a_ref[...], wl2b_ref[...])
    wr2 = wr2_ref[...]
    for bb in range(BS):
        fm = fm5_ref[bb][:, :64]
        br = None
        for i in range(64):
            contrib = fm[:, i:i + 1] * wb2_ref[:, i, :]
            br = contrib if br is None else br + contrib
        br = _lrelu(br)
        root = _dot(fm, wr2)
        for dd in range(DEG):
            o3 = _dot(br[:, dd * 64:(dd + 1) * 64], wlf)
            out_ref[bb, :, dd, :] = o3 + root


def _tree_gcn2(fm5, wr2, wb2, wl2a, wl2b):
    node = NODE1 * DEG
    nblk = 64
    out = pl.pallas_call(
        _tgcn2_body,
        out_shape=jax.ShapeDtypeStruct((BS, node, DEG, 3), _F32),
        grid=(node // nblk,),
        in_specs=[
            pl.BlockSpec((BS, nblk, 128), lambda j: (0, j, 0)),
            pl.BlockSpec((nblk, 64, DEG * 64), lambda j: (j, 0, 0)),
            pl.BlockSpec((64, 3), lambda j: (0, 0)),
            pl.BlockSpec((64, 64 * S), lambda j: (0, 0)),
            pl.BlockSpec((64 * S, 3), lambda j: (0, 0)),
        ],
        out_specs=pl.BlockSpec((BS, nblk, DEG, 3), lambda j: (0, j, 0, 0)),
    )(fm5, wb2, wr2, wl2a, wl2b)
    return out.reshape(BS, node * DEG, 3)


def kernel(coase_points, dir0, w1, b1, dir1, dir2, w3, b3, dir3, w4, b4,
           dir4, w5, b5, dir5, wr1, wb1, wl1a, wl1b, tb1, wr2, wb2, wl2a,
           wl2b):
    pc1 = _stage1(coase_points, dir0, w1, b1, dir1, wr1, wb1, wl1a, wl1b,
                  tb1)
    ni, nd, fm2 = _knn2(pc1, dir2)
    ni_flat = jnp.transpose(ni, (0, 2, 1)).reshape(-1)
    fm3 = _conv_layer2(fm2, ni_flat, nd, w3, b3, dir3)
    fm4 = _conv_layer2(fm3, ni_flat, nd, w4, b4, dir4)
    fm5 = _conv_layer2(fm4, ni_flat, nd, w5, b5, dir5)
    pc2 = _tree_gcn2(fm5, wr2, wb2, wl2a, wl2b)
    return pc1, pc2
